# Initial kernel scaffold; baseline (speedup 1.0000x reference)
import jax
import jax.numpy as jnp
from jax.experimental import pallas as pl


def kernel(x, edge_index, W, b, prelu_w):
    raise NotImplementedError("write your pallas kernel here")


# speedup vs baseline: 21.0974x; 21.0974x over previous
import functools

import jax
import jax.numpy as jnp
from jax import lax
from jax.experimental import pallas as pl
from jax.experimental.pallas import tpu as pltpu
from jax.experimental.pallas import tpu_sc as plsc

N = 10000
E = 320000
D = 128

NC = 2
NS = 16
NW = NC * NS
EPW = E // NW
CH = 80
NCHUNK = EPW // CH
STRIPE = 632
TR = STRIPE * NS

@functools.cache
def _mesh():
    return plsc.VectorSubcoreMesh(
        core_axis_name="c", subcore_axis_name="s",
        num_cores=NC, num_subcores=NS)


@functools.cache
def _hist_kernel():
    return pl.kernel(
        _hist_body,
        out_type=jax.ShapeDtypeStruct((NW, TR), jnp.float32),
        mesh=_mesh(),
        scratch_types=[
            pltpu.VMEM((EPW,), jnp.int32),
            pltpu.VMEM((TR,), jnp.float32),
        ],
        compiler_params=pltpu.CompilerParams(needs_layout_passes=False),
    )


def _hist_body(col_hbm, out_hbm, colbuf, histbuf):
    c = lax.axis_index("c")
    s = lax.axis_index("s")
    wid = s * NC + c

    zvec = jnp.zeros((16,), jnp.float32)
    ones16 = jnp.full((16,), 1.0, jnp.float32)

    def zero16(i, _):
        histbuf[pl.ds(i * 16, 16)] = zvec
        return 0
    lax.fori_loop(0, TR // 16, zero16, 0)

    base = pl.multiple_of(wid * EPW, 8)
    pltpu.sync_copy(col_hbm.at[pl.ds(base, EPW)], colbuf)

    def count16(i, _):
        idx = colbuf[pl.ds(i * 16, 16)]
        plsc.addupdate_scatter(histbuf, [idx], ones16)
        return 0
    lax.fori_loop(0, EPW // 16, count16, 0)

    pltpu.sync_copy(histbuf, out_hbm.at[wid])


@functools.cache
def _agg_kernel():
    return pl.kernel(
        _agg_body,
        out_type=jax.ShapeDtypeStruct((NC, TR, D), jnp.float32),
        mesh=_mesh(),
        scratch_types=[
            pltpu.VMEM((CH,), jnp.int32),
            pltpu.VMEM((CH,), jnp.int32),
            pltpu.VMEM((CH, D), jnp.float32),
            pltpu.VMEM((STRIPE // 4, D), jnp.float32),
            pltpu.VMEM_SHARED((TR, D), jnp.float32),
            pltpu.SemaphoreType.DMA,
        ],
    )


def _agg_body(y_hbm, row_hbm, col_hbm, out_hbm,
              rowbuf, colbuf, rows_v, zbuf, acc, sem):
    c = lax.axis_index("c")
    s = lax.axis_index("s")
    wid = s * NC + c

    zvec = jnp.zeros((16,), jnp.float32)

    def zero_row(i, _):
        for j in range(D // 16):
            zbuf[i, pl.ds(j * 16, 16)] = zvec
        return 0
    lax.fori_loop(0, STRIPE // 4, zero_row, 0)
    for k in range(4):
        pltpu.sync_copy(
            zbuf, acc.at[pl.ds(s * STRIPE + k * (STRIPE // 4), STRIPE // 4)])
    plsc.subcore_barrier()

    def chunk(g, _):
        base = pl.multiple_of(wid * EPW + g * CH, 8)
        pltpu.sync_copy(row_hbm.at[pl.ds(base, CH)], rowbuf)
        pltpu.sync_copy(col_hbm.at[pl.ds(base, CH)], colbuf)
        pltpu.async_copy(y_hbm.at[rowbuf], rows_v, sem).wait()
        pltpu.sync_copy(rows_v, acc.at[colbuf], add=True)
        return 0
    lax.fori_loop(0, NCHUNK, chunk, 0)
    plsc.subcore_barrier()

    pltpu.sync_copy(acc.at[pl.ds(s * STRIPE, STRIPE)],
                    out_hbm.at[c, pl.ds(s * STRIPE, STRIPE)])


_BLK = 1024


def _scale_matmul_body(x_ref, w_ref, h_ref, y_ref):
    deg = 1.0 + jnp.sum(h_ref[...], axis=0)
    dis = lax.rsqrt(deg)
    xl = jnp.dot(x_ref[...], w_ref[...], preferred_element_type=jnp.float32)
    y_ref[...] = xl * dis[:, None]


def _finish_body(p_ref, y_ref, h_ref, b_ref, pw_ref, o_ref):
    deg = 1.0 + jnp.sum(h_ref[...], axis=0)
    dis = lax.rsqrt(deg)
    z = dis[:, None] * (p_ref[0] + p_ref[1] + y_ref[...]) + b_ref[...]
    o_ref[...] = jnp.where(z >= 0, z, pw_ref[...] * z)


def kernel(x, edge_index, W, b, prelu_w):
    row = edge_index[0].astype(jnp.int32)
    col = edge_index[1].astype(jnp.int32)

    hist = _hist_kernel()(col)

    y = pl.pallas_call(
        _scale_matmul_body,
        grid=(pl.cdiv(N, _BLK),),
        in_specs=[
            pl.BlockSpec((_BLK, D), lambda i: (i, 0)),
            pl.BlockSpec((D, D), lambda i: (0, 0)),
            pl.BlockSpec((NW, _BLK), lambda i: (0, i)),
        ],
        out_specs=pl.BlockSpec((_BLK, D), lambda i: (i, 0)),
        out_shape=jax.ShapeDtypeStruct((N, D), jnp.float32),
    )(x, W, hist)

    parts = _agg_kernel()(y, row, col)

    out = pl.pallas_call(
        _finish_body,
        grid=(pl.cdiv(N, _BLK),),
        in_specs=[
            pl.BlockSpec((NC, _BLK, D), lambda i: (0, i, 0)),
            pl.BlockSpec((_BLK, D), lambda i: (i, 0)),
            pl.BlockSpec((NW, _BLK), lambda i: (0, i)),
            pl.BlockSpec((D,), lambda i: (0,)),
            pl.BlockSpec((D,), lambda i: (0,)),
        ],
        out_specs=pl.BlockSpec((_BLK, D), lambda i: (i, 0)),
        out_shape=jax.ShapeDtypeStruct((N, D), jnp.float32),
    )(parts, y, hist, b, prelu_w)

    return out


# speedup vs baseline: 45.6000x; 2.1614x over previous
import functools

import jax
import jax.numpy as jnp
from jax import lax
from jax.experimental import pallas as pl
from jax.experimental.pallas import tpu as pltpu
from jax.experimental.pallas import tpu_sc as plsc

N = 10000
E = 320000
D = 128

NC = 2
NS = 16
NW = NC * NS
EPW = E // NW
STRIPE = 632
TR = STRIPE * NS

@functools.cache
def _mesh():
    return plsc.VectorSubcoreMesh(
        core_axis_name="c", subcore_axis_name="s",
        num_cores=NC, num_subcores=NS)


@functools.cache
def _hist_kernel():
    return pl.kernel(
        _hist_body,
        out_type=jax.ShapeDtypeStruct((NW, TR), jnp.float32),
        mesh=_mesh(),
        scratch_types=[
            pltpu.VMEM((EPW,), jnp.int32),
            pltpu.VMEM((TR,), jnp.float32),
        ],
        compiler_params=pltpu.CompilerParams(needs_layout_passes=False),
    )


def _hist_body(col_hbm, out_hbm, colbuf, histbuf):
    c = lax.axis_index("c")
    s = lax.axis_index("s")
    wid = s * NC + c

    zvec = jnp.zeros((16,), jnp.float32)
    ones16 = jnp.full((16,), 1.0, jnp.float32)

    def zero16(i, _):
        histbuf[pl.ds(i * 16, 16)] = zvec
        return 0
    lax.fori_loop(0, TR // 16, zero16, 0)

    base = pl.multiple_of(wid * EPW, 8)
    pltpu.sync_copy(col_hbm.at[pl.ds(base, EPW)], colbuf)

    def count16(i, _):
        idx = colbuf[pl.ds(i * 16, 16)]
        plsc.addupdate_scatter(histbuf, [idx], ones16)
        return 0
    lax.fori_loop(0, EPW // 16, count16, 0)

    pltpu.sync_copy(histbuf, out_hbm.at[wid])


GW = 128
GCH = 80
EPAD = NW * GCH * GW
NBUF = 2
NHALF = 2
GCH_H = GCH // NHALF


@functools.cache
def _agg_kernel():
    return pl.kernel(
        _agg_body,
        out_type=jax.ShapeDtypeStruct((NC, TR, D), jnp.float32),
        mesh=_mesh(),
        scratch_types=[
            pltpu.VMEM((GCH_H, GW), jnp.int32),
            pltpu.VMEM((GCH_H, GW), jnp.int32),
            pltpu.VMEM((NBUF, GW, D), jnp.float32),
            pltpu.VMEM_SHARED((TR, D), jnp.float32),
            pltpu.SemaphoreType.DMA((NBUF,)),
            pltpu.SemaphoreType.DMA((NBUF,)),
        ],
    )


def _agg_body(y_hbm, row_hbm, col_hbm, out_hbm,
              rowb, colb, ringbuf, acc, gsem, ssem):
    rows = [ringbuf.at[b] for b in range(NBUF)]
    c = lax.axis_index("c")
    s = lax.axis_index("s")
    wid = s * NC + c

    zvec = jnp.zeros((16,), jnp.float32)

    def zero_row(i, _):
        for b in range(NBUF):
            for j in range(D // 16):
                ringbuf[b, i, pl.ds(j * 16, 16)] = zvec
        return 0
    lax.fori_loop(0, GW, zero_row, 0)
    for k in range(4):
        pltpu.sync_copy(rows[0], acc.at[pl.ds(s * STRIPE + k * GW, GW)])
    pltpu.sync_copy(ringbuf.at[1].at[pl.ds(0, 120)],
                    acc.at[pl.ds(s * STRIPE + 4 * GW, 120)])
    plsc.subcore_barrier()

    for h in range(NHALF):
        pltpu.sync_copy(row_hbm.at[wid, pl.ds(h * GCH_H, GCH_H)], rowb)
        pltpu.sync_copy(col_hbm.at[wid, pl.ds(h * GCH_H, GCH_H)], colb)

        for b in range(NBUF):
            pltpu.async_copy(y_hbm.at[rowb.at[b]], rows[b], gsem.at[b])

        def ring(t, _):
            for b in range(NBUF):
                g = t * NBUF + b
                pltpu.make_async_copy(y_hbm.at[rowb.at[g]], rows[b],
                                      gsem.at[b]).wait()
                pltpu.async_copy(rows[b], acc.at[colb.at[g]], ssem.at[b],
                                 add=True)

                @pl.when(g + NBUF < GCH_H)
                def _refill():
                    pltpu.make_async_copy(rows[b], acc.at[colb.at[g]],
                                          ssem.at[b]).wait()
                    pltpu.async_copy(y_hbm.at[rowb.at[g + NBUF]], rows[b],
                                     gsem.at[b])
            return 0
        lax.fori_loop(0, GCH_H // NBUF, ring, 0)

        for b in range(NBUF):
            pltpu.make_async_copy(rows[b], acc.at[colb.at[0]],
                                  ssem.at[b]).wait()
    plsc.subcore_barrier()

    pltpu.sync_copy(acc.at[pl.ds(s * STRIPE, STRIPE)],
                    out_hbm.at[c, pl.ds(s * STRIPE, STRIPE)])


_BLK = 1024


def _scale_matmul_body(x_ref, w_ref, h_ref, y_ref):
    deg = 1.0 + jnp.sum(h_ref[...], axis=0)
    dis = lax.rsqrt(deg)
    xl = jnp.dot(x_ref[...], w_ref[...], preferred_element_type=jnp.float32)
    y_ref[...] = xl * dis[:, None]


def _finish_body(p_ref, y_ref, h_ref, b_ref, pw_ref, o_ref):
    deg = 1.0 + jnp.sum(h_ref[...], axis=0)
    dis = lax.rsqrt(deg)
    z = dis[:, None] * (p_ref[0] + p_ref[1] + y_ref[...]) + b_ref[...]
    o_ref[...] = jnp.where(z >= 0, z, pw_ref[...] * z)


def kernel(x, edge_index, W, b, prelu_w):
    row = edge_index[0].astype(jnp.int32)
    col = edge_index[1].astype(jnp.int32)

    extra = EPAD - E
    pad_i = jnp.arange(extra, dtype=jnp.int32)
    row_p = jnp.concatenate([row, pad_i % N]).reshape(NW, GCH, GW)
    col_p = jnp.concatenate([col, N + pad_i % (TR - N)]).reshape(NW, GCH, GW)

    hist = _hist_kernel()(col)

    y = pl.pallas_call(
        _scale_matmul_body,
        grid=(pl.cdiv(N, _BLK),),
        in_specs=[
            pl.BlockSpec((_BLK, D), lambda i: (i, 0)),
            pl.BlockSpec((D, D), lambda i: (0, 0)),
            pl.BlockSpec((NW, _BLK), lambda i: (0, i)),
        ],
        out_specs=pl.BlockSpec((_BLK, D), lambda i: (i, 0)),
        out_shape=jax.ShapeDtypeStruct((N, D), jnp.float32),
    )(x, W, hist)

    parts = _agg_kernel()(y, row_p, col_p)

    out = pl.pallas_call(
        _finish_body,
        grid=(pl.cdiv(N, _BLK),),
        in_specs=[
            pl.BlockSpec((NC, _BLK, D), lambda i: (0, i, 0)),
            pl.BlockSpec((_BLK, D), lambda i: (i, 0)),
            pl.BlockSpec((NW, _BLK), lambda i: (0, i)),
            pl.BlockSpec((D,), lambda i: (0,)),
            pl.BlockSpec((D,), lambda i: (0,)),
        ],
        out_specs=pl.BlockSpec((_BLK, D), lambda i: (i, 0)),
        out_shape=jax.ShapeDtypeStruct((N, D), jnp.float32),
    )(parts, y, hist, b, prelu_w)

    return out
